# baseline (speedup 1.0000x reference)
import jax
import jax.numpy as jnp
from jax import lax
from jax.experimental import pallas as pl
from jax.experimental.pallas import tpu as pltpu
from jax.experimental.pallas import tpu_sc as plsc

_NC = 2
_NS = 16
_NW = _NC * _NS
_BATCH = 4096
_NF = 26
_BPW = _BATCH // _NW
_NV = 2_600_000


def _body(wp_hbm, xt_hbm, out_hbm, idx_v, rows_v, outf_v, lof_v, sem):
    c = lax.axis_index("c")
    s = lax.axis_index("s")
    wid = s * _NC + c

    pltpu.sync_copy(xt_hbm.at[:, pl.ds(wid * _BPW, _BPW)], idx_v)

    i16 = lax.iota(jnp.int32, 16)

    def floop(f, carry):
        off = f * 100000

        def chunk(g8, carry2):
            vv = idx_v[f, pl.ds(g8 * 16, 16)] + off
            idx_v[f, pl.ds(g8 * 16, 16)] = lax.shift_right_logical(vv, 2)
            lof_v[pl.ds(g8 * 16, 16)] = lax.bitwise_and(vv, 3) * 32
            return carry2

        lax.fori_loop(0, _BPW // 16, chunk, 0)

        pltpu.async_copy(wp_hbm.at[idx_v.at[f]], rows_v, sem).wait()

        def extract(g8, carry2):
            lofs = lof_v[pl.ds(g8 * 16, 16)]
            for kk in range(16):
                r = g8 * 16 + kk
                colv = jnp.full((16,), lofs[kk], jnp.int32) + i16
                rowv = jnp.full((16,), r, jnp.int32)
                g0 = plsc.load_gather(rows_v, [rowv, colv])
                g1 = plsc.load_gather(rows_v, [rowv, colv + 16])
                q, rm = r // 4, (r % 4) * 32
                outf_v[1 + q, pl.ds(rm, 16)] = g0
                outf_v[1 + q, pl.ds(rm + 16, 16)] = g1
            return carry2

        lax.fori_loop(0, _BPW // 16, extract, 0)

        pltpu.sync_copy(
            outf_v.at[pl.ds(1, 32)], out_hbm.at[f, pl.ds(wid * 32, 32), :]
        )
        return carry

    lax.fori_loop(0, _NF, floop, 0)


@jax.jit
def kernel(x, W):
    mesh = plsc.VectorSubcoreMesh(
        core_axis_name="c", subcore_axis_name="s", num_cores=_NC, num_subcores=_NS
    )
    one = W[0, 0] * 1e-38 + 1.0
    wp = lax.optimization_barrier(W.reshape(_NV // 4, 128) * one)
    xt = x.T
    out = pl.kernel(
        _body,
        out_type=jax.ShapeDtypeStruct((_NF, _BATCH * 32 // 128, 128), jnp.float32),
        mesh=mesh,
        scratch_types=[
            pltpu.VMEM((_NF, _BPW), jnp.int32),
            pltpu.VMEM((_BPW, 128), jnp.float32),
            pltpu.VMEM((33, 128), jnp.float32),
            pltpu.VMEM((128,), jnp.int32),
            pltpu.SemaphoreType.DMA,
        ],
        compiler_params=pltpu.CompilerParams(use_tc_tiling_on_sc=False, needs_layout_passes=False),
    )(wp, xt)
    return out.reshape(_NF, _BATCH, 32).transpose(1, 0, 2)
